# Initial kernel scaffold; baseline (speedup 1.0000x reference)
import jax
import jax.numpy as jnp
from jax.experimental import pallas as pl


def kernel(x, edge_index, W1, W2):
    raise NotImplementedError("write your pallas kernel here")


# speedup vs baseline: 6.9372x; 6.9372x over previous
import functools

import jax
import jax.numpy as jnp
from jax import lax
from jax.experimental import pallas as pl
from jax.experimental.pallas import tpu as pltpu
from jax.experimental.pallas import tpu_sc as plsc

N = 10000
FEAT = 128
EMBED = 128
NUM_CLASSES = 64

NC = 2
NS = 16
NW = NC * NS
CHUNK = 128
NPAD = 10112
ZROWS = NPAD // NS
OROWS = 632
OTAIL = N - 15 * OROWS


def _deg_kernel(ep_total):
    chunks_per_tile = ep_total // (NW * CHUNK)
    e_per_tile = chunks_per_tile * CHUNK
    mesh = plsc.VectorSubcoreMesh(core_axis_name="c", subcore_axis_name="s")

    @functools.partial(
        pl.kernel,
        mesh=mesh,
        out_type=jax.ShapeDtypeStruct((NC, N, 16), jnp.float32),
        scratch_types=[
            pltpu.VMEM((CHUNK,), jnp.int32),
            pltpu.VMEM((CHUNK, 16), jnp.float32),
            pltpu.VMEM_SHARED((NPAD, 16), jnp.float32),
        ],
        compiler_params=pltpu.CompilerParams(use_tc_tiling_on_sc=False),
    )
    def k(dst_hbm, ones_hbm, z_hbm, out_hbm, didx_v, ones_v, deg_sh):
        c = lax.axis_index("c")
        s = lax.axis_index("s")
        wid = c * NS + s
        base = wid * e_per_tile
        pltpu.sync_copy(z_hbm.at[pl.ds(s * ZROWS, ZROWS)],
                        deg_sh.at[pl.ds(s * ZROWS, ZROWS)])
        pltpu.sync_copy(ones_hbm, ones_v)
        plsc.subcore_barrier()

        @pl.loop(0, chunks_per_tile)
        def _(j):
            pltpu.sync_copy(dst_hbm.at[pl.ds(base + j * CHUNK, CHUNK)], didx_v)
            pltpu.sync_copy(ones_v, deg_sh.at[didx_v], add=True)

        plsc.subcore_barrier()
        _copy_out(deg_sh, out_hbm, c, s)

    return k


def _copy_out(table_sh, out_hbm, c, s):
    @pl.when(s < NS - 1)
    def _():
        pltpu.sync_copy(table_sh.at[pl.ds(s * OROWS, OROWS)],
                        out_hbm.at[c, pl.ds(s * OROWS, OROWS)])

    @pl.when(s == NS - 1)
    def _():
        pltpu.sync_copy(table_sh.at[pl.ds((NS - 1) * OROWS, OTAIL)],
                        out_hbm.at[c, pl.ds((NS - 1) * OROWS, OTAIL)])


def _seg_sum_kernel(width, ep_total):
    chunks_per_tile = ep_total // (NW * CHUNK)
    e_per_tile = chunks_per_tile * CHUNK
    mesh = plsc.VectorSubcoreMesh(core_axis_name="c", subcore_axis_name="s")

    @functools.partial(
        pl.kernel,
        mesh=mesh,
        out_type=jax.ShapeDtypeStruct((NC, N, width), jnp.float32),
        scratch_types=[
            pltpu.VMEM((CHUNK,), jnp.int32),
            pltpu.VMEM((CHUNK,), jnp.int32),
            pltpu.VMEM((CHUNK, width), jnp.float32),
            pltpu.VMEM_SHARED((NPAD, width), jnp.float32),
            pltpu.SemaphoreType.DMA,
        ],
        compiler_params=pltpu.CompilerParams(use_tc_tiling_on_sc=False),
    )
    def k(h_hbm, src_hbm, dst_hbm, z_hbm, out_hbm,
          sidx_v, didx_v, rows_v, agg_sh, sem):
        c = lax.axis_index("c")
        s = lax.axis_index("s")
        wid = c * NS + s
        base = wid * e_per_tile
        pltpu.sync_copy(z_hbm.at[pl.ds(s * ZROWS, ZROWS)],
                        agg_sh.at[pl.ds(s * ZROWS, ZROWS)])
        plsc.subcore_barrier()

        @pl.loop(0, chunks_per_tile)
        def _(j):
            e0 = base + j * CHUNK
            pltpu.sync_copy(src_hbm.at[pl.ds(e0, CHUNK)], sidx_v)
            pltpu.sync_copy(dst_hbm.at[pl.ds(e0, CHUNK)], didx_v)
            pltpu.async_copy(h_hbm.at[sidx_v], rows_v, sem).wait()
            pltpu.sync_copy(rows_v, agg_sh.at[didx_v], add=True)

        plsc.subcore_barrier()
        _copy_out(agg_sh, out_hbm, c, s)

    return k


def _dis_block(degp_ref):
    deg = degp_ref[0, :, 0:1] + degp_ref[1, :, 0:1]
    return lax.rsqrt(jnp.maximum(deg, 1.0))


_TC_R = 2000


def _tc1_body(x_ref, w_ref, degp_ref, out_ref):
    dis = _dis_block(degp_ref)
    h = jnp.dot(x_ref[...], w_ref[...], preferred_element_type=jnp.float32)
    out_ref[...] = h * dis


def _tc2_body(aggp_ref, degp_ref, w_ref, out_ref):
    dis = _dis_block(degp_ref)
    agg = aggp_ref[0] + aggp_ref[1]
    h = jnp.maximum(agg * dis, 0.0)
    out_ref[...] = jnp.dot(h, w_ref[...],
                           preferred_element_type=jnp.float32) * dis


def _tc3_body(aggp_ref, degp_ref, out_ref):
    dis = _dis_block(degp_ref)
    o = (aggp_ref[0] + aggp_ref[1]) * dis
    m = jnp.max(o, axis=1, keepdims=True)
    lse = jnp.log(jnp.sum(jnp.exp(o - m), axis=1, keepdims=True))
    out_ref[...] = o - m - lse


def _deg_spec():
    return pl.BlockSpec((NC, _TC_R, 16), lambda i: (0, i, 0))


def _tc1_call(x, W1, degp):
    return pl.pallas_call(
        _tc1_body,
        grid=(N // _TC_R,),
        in_specs=[
            pl.BlockSpec((_TC_R, FEAT), lambda i: (i, 0)),
            pl.BlockSpec((FEAT, EMBED), lambda i: (0, 0)),
            _deg_spec(),
        ],
        out_specs=pl.BlockSpec((_TC_R, EMBED), lambda i: (i, 0)),
        out_shape=jax.ShapeDtypeStruct((N, EMBED), jnp.float32),
    )(x, W1, degp)


def _tc2_call(aggp, degp, W2):
    return pl.pallas_call(
        _tc2_body,
        grid=(N // _TC_R,),
        in_specs=[
            pl.BlockSpec((NC, _TC_R, EMBED), lambda i: (0, i, 0)),
            _deg_spec(),
            pl.BlockSpec((EMBED, NUM_CLASSES), lambda i: (0, 0)),
        ],
        out_specs=pl.BlockSpec((_TC_R, NUM_CLASSES), lambda i: (i, 0)),
        out_shape=jax.ShapeDtypeStruct((N, NUM_CLASSES), jnp.float32),
    )(aggp, degp, W2)


def _tc3_call(aggp, degp):
    return pl.pallas_call(
        _tc3_body,
        grid=(N // _TC_R,),
        in_specs=[
            pl.BlockSpec((NC, _TC_R, NUM_CLASSES), lambda i: (0, i, 0)),
            _deg_spec(),
        ],
        out_specs=pl.BlockSpec((_TC_R, NUM_CLASSES), lambda i: (i, 0)),
        out_shape=jax.ShapeDtypeStruct((N, NUM_CLASSES), jnp.float32),
    )(aggp, degp)


def kernel(x, edge_index, W1, W2):
    src = edge_index[0]
    dst = edge_index[1]
    e = src.shape[0]
    ep_total = -(-e // (NW * CHUNK)) * (NW * CHUNK)
    pad = ep_total - e
    srcp = jnp.concatenate([src, jnp.zeros((pad,), jnp.int32)])
    dstp = jnp.concatenate([dst, jnp.full((pad,), N, jnp.int32)])
    ones16 = jnp.ones((CHUNK, 16), jnp.float32)
    z16 = jnp.zeros((NPAD, 16), jnp.float32)
    z_embed = jnp.zeros((NPAD, EMBED), jnp.float32)
    z_cls = jnp.zeros((NPAD, NUM_CLASSES), jnp.float32)

    degp = _deg_kernel(ep_total)(dstp, ones16, z16)
    h1s = _tc1_call(x, W1, degp)
    agg1 = _seg_sum_kernel(EMBED, ep_total)(h1s, srcp, dstp, z_embed)
    h2s = _tc2_call(agg1, degp, W2)
    agg2 = _seg_sum_kernel(NUM_CLASSES, ep_total)(h2s, srcp, dstp, z_cls)
    return _tc3_call(agg2, degp)
